# baseline (speedup 1.0000x reference)
import functools

import jax
import jax.numpy as jnp
from jax import lax
from jax.experimental import pallas as pl
from jax.experimental.pallas import tpu as pltpu
from jax.experimental.pallas import tpu_sc as plsc

_LW = 128
_NSLICE = 2
_TN = 4096


def _tc_prepare(w_param, b_param, w2, b_out):
    h, p = w_param.shape[1], w_param.shape[1]
    d = w2.shape[0]
    p = w_param.shape[1]

    def body(wp_ref, bp_ref, w2_ref, bo_ref, wc_ref, be_ref):
        w2v = w2_ref[...]
        dn_t = (((1,), (1,)), ((), ()))
        be_ref[...] = lax.dot_general(bp_ref[...], w2v, dn_t,
                                      preferred_element_type=jnp.float32) + bo_ref[...]
        wc_ref[...] = lax.dot_general(wp_ref[...], w2v,
                                      (((0,), (1,)), ((), ())),
                                      preferred_element_type=jnp.float32)

    return pl.pallas_call(
        body,
        out_shape=(
            jax.ShapeDtypeStruct((p, d), jnp.float32),
            jax.ShapeDtypeStruct((1, d), jnp.float32),
        ),
    )(w_param, b_param, w2, b_out)


def _sc_gather(ids3d, table):
    nw_dim, idxrows_per_w, lw = ids3d.shape
    v, h = table.shape
    n = nw_dim * idxrows_per_w * lw
    info = plsc.get_sparse_core_info()
    nw = info.num_cores * info.num_subcores
    assert nw == nw_dim
    ch = 5
    nch = idxrows_per_w // ch
    rows_per_chunk = ch * lw
    rows_per_w = idxrows_per_w * lw
    assert nch * ch == idxrows_per_w

    mesh = plsc.VectorSubcoreMesh(core_axis_name="c", subcore_axis_name="s")

    @functools.partial(
        pl.kernel,
        out_type=jax.ShapeDtypeStruct((n, h), jnp.float32),
        mesh=mesh,
        scratch_types=[
            pltpu.VMEM((idxrows_per_w, lw), jnp.int32),
            pltpu.VMEM((rows_per_chunk, h), jnp.float32),
            pltpu.SemaphoreType.DMA,
        ],
        compiler_params=pltpu.CompilerParams(use_tc_tiling_on_sc=False),
    )
    def k(ids_hbm, table_hbm, out_hbm, idx_v, rows_v, sem):
        wid = lax.axis_index("s") * info.num_cores + lax.axis_index("c")
        row_base = wid * rows_per_w
        pltpu.sync_copy(ids_hbm.at[wid], idx_v)

        def body(c, carry):
            copies = [
                pltpu.async_copy(
                    table_hbm.at[idx_v.at[c * ch + j]],
                    rows_v.at[pl.ds(j * lw, lw)],
                    sem,
                )
                for j in range(ch)
            ]
            for cp in copies:
                cp.wait()
            out_off = pl.multiple_of(row_base + c * rows_per_chunk, 8)
            pltpu.sync_copy(rows_v, out_hbm.at[pl.ds(out_off, rows_per_chunk)])
            return carry

        lax.fori_loop(0, nch, body, 0)

    return k(ids3d, table)


def _tc_combine_slice(gathered_s, params_2d, w1, wc_t, b_eff, prev, s, n, tn=_TN):
    ns, h = gathered_s.shape
    p = wc_t.shape[0]
    d = wc_t.shape[1]
    nblk = ns // tn
    blk0 = s * nblk
    assert nblk * tn == ns

    def body(g_ref, pk_ref, w1_ref, wc_ref, be_ref, *o_refs):
        o_ref = o_refs[-1]
        emb = lax.dot_general(g_ref[...], w1_ref[...], (((1,), (1,)), ((), ())),
                              preferred_element_type=jnp.float32)
        pe = lax.dot_general(pk_ref[...], wc_ref[...], (((1,), (0,)), ((), ())),
                             preferred_element_type=jnp.float32)
        o_ref[...] = emb + pe + be_ref[...]

    in_specs = [
        pl.BlockSpec((tn, h), lambda i: (i, 0)),
        pl.BlockSpec((tn, p), lambda i: (blk0 + i, 0)),
        pl.BlockSpec((d, h), lambda i: (0, 0)),
        pl.BlockSpec((p, d), lambda i: (0, 0)),
        pl.BlockSpec((1, d), lambda i: (0, 0)),
    ]
    args = [gathered_s, params_2d, w1, wc_t, b_eff]
    aliases = {}
    if prev is not None:
        in_specs.append(pl.BlockSpec(memory_space=pl.ANY))
        args.append(prev)
        aliases = {5: 0}

    return pl.pallas_call(
        body,
        grid=(nblk,),
        in_specs=in_specs,
        out_specs=pl.BlockSpec((tn, d), lambda i: (blk0 + i, 0)),
        out_shape=jax.ShapeDtypeStruct((n, d), jnp.float32),
        input_output_aliases=aliases,
    )(*args)


def kernel(propensity_type_ids, propensity_params, type_table, W_param, b_param, W_out, b_out):
    b, r = propensity_type_ids.shape
    _, _, p = propensity_params.shape
    v, h = type_table.shape
    d = W_out.shape[0]
    n = b * r
    w1 = W_out[:, :h]
    w2 = W_out[:, h:]
    wc_t, b_eff = _tc_prepare(W_param, b_param.reshape(1, h), w2, b_out.reshape(1, d))
    info = plsc.get_sparse_core_info()
    nw = info.num_cores * info.num_subcores
    ns = n // _NSLICE
    ids4d = propensity_type_ids.reshape(
        _NSLICE, nw, ns // (nw * _LW), _LW
    ).astype(jnp.int32)
    params_2d = propensity_params.reshape(n, p)

    gathered = [_sc_gather(ids4d[s], type_table) for s in range(_NSLICE)]
    out = None
    for s in range(_NSLICE):
        out = _tc_combine_slice(gathered[s], params_2d, w1, wc_t, b_eff, out, s, n)
    return out.reshape(b, r, d)
